# baseline (speedup 1.0000x reference)
import functools

import jax
import jax.numpy as jnp
from jax import lax
from jax.experimental import pallas as pl
from jax.experimental.pallas import tpu as pltpu
from jax.experimental.pallas import tpu_sc as plsc

N = 8
C = 192
H = 56
W = 56
E = 16
D = 64
K = 4
S = H * W
NHW = N * S
S_BLK = 128

NW = 32
P = NHW // NW
CL = 28
RC = CL * K
NCH = P // CL
NBUF = 2


def _gate_kernel(x_ref, w_ref, b_ref, rid_ref, wts_ref):
    n = pl.program_id(0)
    sb = pl.program_id(1)
    x = x_ref[0]
    logits = jnp.dot(w_ref[...], x,
                     preferred_element_type=jnp.float32) + b_ref[...]
    m = jnp.max(logits, axis=0, keepdims=True)
    p = jnp.exp(logits - m)
    probs = p / jnp.sum(p, axis=0, keepdims=True)
    pt = probs.T

    iota_e = jax.lax.broadcasted_iota(jnp.int32, pt.shape, 1)
    s_col = (jax.lax.broadcasted_iota(jnp.int32, (S_BLK, 1), 0)
             + sb * S_BLK)
    base = n * E
    wp = pt
    rids = []
    vals = []
    for _ in range(K):
        mx = jnp.max(wp, axis=1, keepdims=True)
        sel_idx = jnp.min(jnp.where(wp == mx, iota_e, E), axis=1,
                          keepdims=True)
        rids.append((base + sel_idx) * S + s_col)
        vals.append(mx)
        wp = jnp.where(iota_e == sel_idx, -jnp.inf, wp)
    rid_ref[0] = jnp.concatenate(rids, axis=1)
    wts_ref[0] = jnp.concatenate(vals, axis=1)


def _gate(xs, gate_w, b2):
    return pl.pallas_call(
        _gate_kernel,
        grid=(N, pl.cdiv(S, S_BLK)),
        in_specs=[
            pl.BlockSpec((1, C, S_BLK), lambda n, s: (n, 0, s)),
            pl.BlockSpec((E, C), lambda n, s: (0, 0)),
            pl.BlockSpec((E, 1), lambda n, s: (0, 0)),
        ],
        out_specs=[
            pl.BlockSpec((1, S_BLK, K), lambda n, s: (n, s, 0)),
            pl.BlockSpec((1, S_BLK, K), lambda n, s: (n, s, 0)),
        ],
        out_shape=[
            jax.ShapeDtypeStruct((N, S, K), jnp.int32),
            jax.ShapeDtypeStruct((N, S, K), jnp.float32),
        ],
    )(xs, gate_w, b2)


def _bcast_lane(v, lane):
    idx = jnp.full((16, 1), lane, jnp.int32)
    return lax.gather(
        v, idx,
        dimension_numbers=lax.GatherDimensionNumbers(
            offset_dims=(), collapsed_slice_dims=(0,),
            start_index_map=(0,)),
        slice_sizes=(1,),
        mode=lax.GatherScatterMode.PROMISE_IN_BOUNDS)


_MESH = plsc.VectorSubcoreMesh(core_axis_name="c", subcore_axis_name="s")


@functools.partial(
    pl.kernel,
    mesh=_MESH,
    out_type=jax.ShapeDtypeStruct((NHW * D,), jnp.float32),
    scratch_types=[
        pltpu.VMEM((P * K,), jnp.int32),
        pltpu.VMEM((P * K,), jnp.float32),
        pltpu.VMEM((NBUF, RC, D), jnp.float32),
        pltpu.VMEM((P * D,), jnp.float32),
        pltpu.SemaphoreType.DMA,
        pltpu.SemaphoreType.DMA,
    ],
    compiler_params=pltpu.CompilerParams(use_tc_tiling_on_sc=False),
)
def _sc_gather(ex_hbm, rid_hbm, w_hbm, out_hbm, rid_v, w_v, gbuf, out_v,
               gsem0, gsem1):
    sems = [gsem0, gsem1]
    wid = lax.axis_index("s") * 2 + lax.axis_index("c")
    base = wid * P
    pltpu.sync_copy(rid_hbm.at[pl.ds(base * K, P * K)], rid_v)
    pltpu.sync_copy(w_hbm.at[pl.ds(base * K, P * K)], w_v)

    def gather_chunk(c, b):
        off = pl.multiple_of(c * RC, 8)
        pltpu.async_copy(ex_hbm.at[rid_v.at[pl.ds(off, RC)]], gbuf.at[b],
                         sems[b])

    def wait_chunk(c, b):
        off = pl.multiple_of(c * RC, 8)
        pltpu.make_async_copy(ex_hbm.at[rid_v.at[pl.ds(off, RC)]],
                              gbuf.at[b], sems[b]).wait()

    def compute_chunk(c, b):
        wait_chunk(c, b)
        for i0 in range(0, CL, 4):
            wv = w_v[pl.ds((c * CL + i0) * K, 16)]
            for ii in range(4):
                i = i0 + ii
                wbs = [_bcast_lane(wv, ii * K + j) for j in range(K)]
                for d in range(D // 16):
                    acc = wbs[0] * gbuf[b, K * i, pl.ds(d * 16, 16)]
                    for j in range(1, K):
                        acc = acc + wbs[j] * gbuf[b, K * i + j,
                                                  pl.ds(d * 16, 16)]
                    out_v[pl.ds((c * CL + i) * D + d * 16, 16)] = acc

    for b in range(NBUF):
        gather_chunk(b, b)

    def loop_body(g, carry):
        for b in range(NBUF):
            c = g * NBUF + b
            compute_chunk(c, b)

            @pl.when(c + NBUF < NCH)
            def _():
                gather_chunk(c + NBUF, b)
        return carry

    lax.fori_loop(0, NCH // NBUF, loop_body, 0)
    pltpu.sync_copy(out_v, out_hbm.at[pl.ds(base * D, P * D)])


@jax.jit
def kernel(x, experts, gate_w, gate_b):
    xs = x.reshape(N, C, S)
    b2 = gate_b.reshape(E, 1)
    rid, wts = _gate(xs, gate_w, b2)
    ex_flat = experts.reshape(N * E * S, D)
    out = _sc_gather(ex_flat, rid.reshape(NHW * K), wts.reshape(NHW * K))
    return out.reshape(N, H, W, D)


# speedup vs baseline: 2.1831x; 2.1831x over previous
import jax
import jax.numpy as jnp
from jax.experimental import pallas as pl

E = 16
D = 64
K = 4
S_BLK = 128


def _moe_block_kernel(x_ref, w_ref, b_ref, ex_ref, out_ref):
    x = x_ref[0]
    logits = jnp.dot(w_ref[...], x,
                     preferred_element_type=jnp.float32) + b_ref[...]
    m = jnp.max(logits, axis=0, keepdims=True)
    p = jnp.exp(logits - m)
    probs = p / jnp.sum(p, axis=0, keepdims=True)

    iota_e = jax.lax.broadcasted_iota(jnp.int32, probs.shape, 0)
    mask = jnp.zeros(probs.shape, jnp.bool_)
    wp = probs
    for _ in range(K):
        mx = jnp.max(wp, axis=0, keepdims=True)
        sel_idx = jnp.min(jnp.where(wp == mx, iota_e, E), axis=0,
                          keepdims=True)
        sel = iota_e == sel_idx
        mask = jnp.logical_or(mask, sel)
        wp = jnp.where(sel, -jnp.inf, wp)
    pm0 = jnp.where(mask, probs, 0.0)
    eye_e = (jax.lax.broadcasted_iota(jnp.int32, (E, E), 0) ==
             jax.lax.broadcasted_iota(jnp.int32, (E, E), 1)
             ).astype(jnp.float32)
    pm = jax.lax.dot_general(pm0, eye_e, (((0,), (0,)), ((), ())),
                             preferred_element_type=jnp.float32)

    acc = pm[:, 0:1] * ex_ref[0, 0]
    for e in range(1, E):
        acc = acc + pm[:, e:e + 1] * ex_ref[0, e]
    out_ref[0] = acc


@jax.jit
def kernel(x, experts, gate_w, gate_b):
    N, C, H, W = x.shape
    S = H * W
    xs = x.reshape(N, C, S)
    exs = experts.reshape(N, E, S, D)
    b2 = gate_b.reshape(E, 1)
    grid = (N, pl.cdiv(S, S_BLK))
    out = pl.pallas_call(
        _moe_block_kernel,
        grid=grid,
        in_specs=[
            pl.BlockSpec((1, C, S_BLK), lambda n, s: (n, 0, s)),
            pl.BlockSpec((E, C), lambda n, s: (0, 0)),
            pl.BlockSpec((E, 1), lambda n, s: (0, 0)),
            pl.BlockSpec((1, E, S_BLK, D), lambda n, s: (n, 0, s, 0)),
        ],
        out_specs=pl.BlockSpec((1, S_BLK, D), lambda n, s: (n, s, 0)),
        out_shape=jax.ShapeDtypeStruct((N, S, D), jnp.float32),
    )(xs, gate_w, b2, exs)
    return out.reshape(N, H, W, D)


# speedup vs baseline: 2.9791x; 1.3646x over previous
import jax
import jax.numpy as jnp
from jax.experimental import pallas as pl

E = 16
D = 64
K = 4
S_BLK = 256


def _moe_block_kernel(x_ref, w_ref, b_ref, ex_ref, out_ref):
    x = x_ref[0]
    logits = jnp.dot(w_ref[...], x,
                     preferred_element_type=jnp.float32) + b_ref[...]
    m = jnp.max(logits, axis=0, keepdims=True)
    p = jnp.exp(logits - m)
    probs = p / jnp.sum(p, axis=0, keepdims=True)

    iota_e = jax.lax.broadcasted_iota(jnp.int32, probs.shape, 0)
    mask = jnp.zeros(probs.shape, jnp.bool_)
    wp = probs
    for _ in range(K):
        mx = jnp.max(wp, axis=0, keepdims=True)
        sel_idx = jnp.min(jnp.where(wp == mx, iota_e, E), axis=0,
                          keepdims=True)
        sel = iota_e == sel_idx
        mask = jnp.logical_or(mask, sel)
        wp = jnp.where(sel, -jnp.inf, wp)
    pm0 = jnp.where(mask, probs, 0.0)
    eye_e = (jax.lax.broadcasted_iota(jnp.int32, (E, E), 0) ==
             jax.lax.broadcasted_iota(jnp.int32, (E, E), 1)
             ).astype(jnp.float32)
    pm = jax.lax.dot_general(pm0, eye_e, (((0,), (0,)), ((), ())),
                             preferred_element_type=jnp.float32)

    acc = pm[:, 0:1] * ex_ref[0, 0]
    for e in range(1, E):
        acc = acc + pm[:, e:e + 1] * ex_ref[0, e]
    out_ref[0] = acc


@jax.jit
def kernel(x, experts, gate_w, gate_b):
    N, C, H, W = x.shape
    S = H * W
    xs = x.reshape(N, C, S)
    exs = experts.reshape(N, E, S, D)
    b2 = gate_b.reshape(E, 1)
    grid = (N, pl.cdiv(S, S_BLK))
    out = pl.pallas_call(
        _moe_block_kernel,
        grid=grid,
        in_specs=[
            pl.BlockSpec((1, C, S_BLK), lambda n, s: (n, 0, s)),
            pl.BlockSpec((E, C), lambda n, s: (0, 0)),
            pl.BlockSpec((E, 1), lambda n, s: (0, 0)),
            pl.BlockSpec((1, E, S_BLK, D), lambda n, s: (n, 0, s, 0)),
        ],
        out_specs=pl.BlockSpec((1, S_BLK, D), lambda n, s: (n, s, 0)),
        out_shape=jax.ShapeDtypeStruct((N, S, D), jnp.float32),
    )(xs, gate_w, b2, exs)
    return out.reshape(N, H, W, D)


# speedup vs baseline: 3.6570x; 1.2275x over previous
import jax
import jax.numpy as jnp
from jax.experimental import pallas as pl

E = 16
D = 64
K = 4
S_BLK = 512


def _moe_block_kernel(x_ref, w_ref, b_ref, ex_ref, out_ref):
    x = x_ref[0]
    logits = jnp.dot(w_ref[...], x,
                     preferred_element_type=jnp.float32) + b_ref[...]
    m = jnp.max(logits, axis=0, keepdims=True)
    p = jnp.exp(logits - m)
    probs = p / jnp.sum(p, axis=0, keepdims=True)

    iota_e = jax.lax.broadcasted_iota(jnp.int32, probs.shape, 0)
    mask = jnp.zeros(probs.shape, jnp.bool_)
    wp = probs
    for _ in range(K):
        mx = jnp.max(wp, axis=0, keepdims=True)
        sel_idx = jnp.min(jnp.where(wp == mx, iota_e, E), axis=0,
                          keepdims=True)
        sel = iota_e == sel_idx
        mask = jnp.logical_or(mask, sel)
        wp = jnp.where(sel, -jnp.inf, wp)
    pm0 = jnp.where(mask, probs, 0.0)
    eye_e = (jax.lax.broadcasted_iota(jnp.int32, (E, E), 0) ==
             jax.lax.broadcasted_iota(jnp.int32, (E, E), 1)
             ).astype(jnp.float32)
    pm = jax.lax.dot_general(pm0, eye_e, (((0,), (0,)), ((), ())),
                             preferred_element_type=jnp.float32)

    acc = pm[:, 0:1] * ex_ref[0, 0]
    for e in range(1, E):
        acc = acc + pm[:, e:e + 1] * ex_ref[0, e]
    out_ref[0] = acc


@jax.jit
def kernel(x, experts, gate_w, gate_b):
    N, C, H, W = x.shape
    S = H * W
    xs = x.reshape(N, C, S)
    exs = experts.reshape(N, E, S, D)
    b2 = gate_b.reshape(E, 1)
    grid = (N, pl.cdiv(S, S_BLK))
    out = pl.pallas_call(
        _moe_block_kernel,
        grid=grid,
        in_specs=[
            pl.BlockSpec((1, C, S_BLK), lambda n, s: (n, 0, s)),
            pl.BlockSpec((E, C), lambda n, s: (0, 0)),
            pl.BlockSpec((E, 1), lambda n, s: (0, 0)),
            pl.BlockSpec((1, E, S_BLK, D), lambda n, s: (n, 0, s, 0)),
        ],
        out_specs=pl.BlockSpec((1, S_BLK, D), lambda n, s: (n, s, 0)),
        out_shape=jax.ShapeDtypeStruct((N, S, D), jnp.float32),
    )(xs, gate_w, b2, exs)
    return out.reshape(N, H, W, D)


# speedup vs baseline: 3.7492x; 1.0252x over previous
import jax
import jax.numpy as jnp
from jax.experimental import pallas as pl

E = 16
D = 64
K = 4
S_BLK = 1024


def _moe_block_kernel(x_ref, w_ref, b_ref, ex_ref, out_ref):
    x = x_ref[0]
    logits = jnp.dot(w_ref[...], x,
                     preferred_element_type=jnp.float32) + b_ref[...]
    m = jnp.max(logits, axis=0, keepdims=True)
    p = jnp.exp(logits - m)
    probs = p / jnp.sum(p, axis=0, keepdims=True)

    iota_e = jax.lax.broadcasted_iota(jnp.int32, probs.shape, 0)
    mask = jnp.zeros(probs.shape, jnp.bool_)
    wp = probs
    for _ in range(K):
        mx = jnp.max(wp, axis=0, keepdims=True)
        sel_idx = jnp.min(jnp.where(wp == mx, iota_e, E), axis=0,
                          keepdims=True)
        sel = iota_e == sel_idx
        mask = jnp.logical_or(mask, sel)
        wp = jnp.where(sel, -jnp.inf, wp)
    pm0 = jnp.where(mask, probs, 0.0)
    eye_e = (jax.lax.broadcasted_iota(jnp.int32, (E, E), 0) ==
             jax.lax.broadcasted_iota(jnp.int32, (E, E), 1)
             ).astype(jnp.float32)
    pm = jax.lax.dot_general(pm0, eye_e, (((0,), (0,)), ((), ())),
                             preferred_element_type=jnp.float32)

    acc = pm[:, 0:1] * ex_ref[0, 0]
    for e in range(1, E):
        acc = acc + pm[:, e:e + 1] * ex_ref[0, e]
    out_ref[0] = acc


@jax.jit
def kernel(x, experts, gate_w, gate_b):
    N, C, H, W = x.shape
    S = H * W
    xs = x.reshape(N, C, S)
    exs = experts.reshape(N, E, S, D)
    b2 = gate_b.reshape(E, 1)
    grid = (N, pl.cdiv(S, S_BLK))
    out = pl.pallas_call(
        _moe_block_kernel,
        grid=grid,
        in_specs=[
            pl.BlockSpec((1, C, S_BLK), lambda n, s: (n, 0, s)),
            pl.BlockSpec((E, C), lambda n, s: (0, 0)),
            pl.BlockSpec((E, 1), lambda n, s: (0, 0)),
            pl.BlockSpec((1, E, S_BLK, D), lambda n, s: (n, 0, s, 0)),
        ],
        out_specs=pl.BlockSpec((1, S_BLK, D), lambda n, s: (n, s, 0)),
        out_shape=jax.ShapeDtypeStruct((N, S, D), jnp.float32),
    )(xs, gate_w, b2, exs)
    return out.reshape(N, H, W, D)


# speedup vs baseline: 4.8271x; 1.2875x over previous
import jax
import jax.numpy as jnp
from jax.experimental import pallas as pl

E = 16
D = 64
K = 4
S_BLK = 784


def _gate_kernel(x_ref, w_ref, b_ref, pm_ref):
    x = x_ref[0]
    logits = jax.lax.dot_general(
        w_ref[...], x, (((1,), (1,)), ((), ())),
        preferred_element_type=jnp.float32) + b_ref[...]
    m = jnp.max(logits, axis=0, keepdims=True)
    p = jnp.exp(logits - m)
    probs = p / jnp.sum(p, axis=0, keepdims=True)

    iota_e = jax.lax.broadcasted_iota(jnp.int32, probs.shape, 0)
    mask = jnp.zeros(probs.shape, jnp.bool_)
    wp = probs
    for _ in range(K):
        mx = jnp.max(wp, axis=0, keepdims=True)
        sel_idx = jnp.min(jnp.where(wp == mx, iota_e, E), axis=0,
                          keepdims=True)
        sel = iota_e == sel_idx
        mask = jnp.logical_or(mask, sel)
        wp = jnp.where(sel, -jnp.inf, wp)
    pm_ref[0] = jnp.where(mask, probs, 0.0)


def _sum_kernel(pm_ref, ex_ref, out_ref):
    pm = pm_ref[0]
    acc = pm[:, 0:1] * ex_ref[0, 0]
    for e in range(1, E):
        acc = acc + pm[:, e:e + 1] * ex_ref[0, e]
    out_ref[0] = acc


@jax.jit
def kernel(x, experts, gate_w, gate_b):
    N, C, H, W = x.shape
    S = H * W
    xs = jnp.transpose(x, (0, 2, 3, 1)).reshape(N, S, C)
    exs = experts.reshape(N, E, S, D)
    b2 = gate_b.reshape(E, 1)

    pm0 = pl.pallas_call(
        _gate_kernel,
        grid=(N,),
        in_specs=[
            pl.BlockSpec((1, S, C), lambda n: (n, 0, 0)),
            pl.BlockSpec((E, C), lambda n: (0, 0)),
            pl.BlockSpec((E, 1), lambda n: (0, 0)),
        ],
        out_specs=pl.BlockSpec((1, E, S), lambda n: (n, 0, 0)),
        out_shape=jax.ShapeDtypeStruct((N, E, S), jnp.float32),
    )(xs, gate_w, b2)

    pmt = jnp.transpose(pm0, (0, 2, 1))

    grid = (N, S // S_BLK)
    out = pl.pallas_call(
        _sum_kernel,
        grid=grid,
        in_specs=[
            pl.BlockSpec((1, S_BLK, E), lambda n, s: (n, s, 0)),
            pl.BlockSpec((1, E, S_BLK, D), lambda n, s: (n, 0, s, 0)),
        ],
        out_specs=pl.BlockSpec((1, S_BLK, D), lambda n, s: (n, s, 0)),
        out_shape=jax.ShapeDtypeStruct((N, S, D), jnp.float32),
    )(pmt, exs)
    return out.reshape(N, H, W, D)


# speedup vs baseline: 5.1319x; 1.0631x over previous
import jax
import jax.numpy as jnp
from jax.experimental import pallas as pl

E = 16
D = 64
K = 4
S_BLK = 1568


def _gate_kernel(x_ref, w_ref, b_ref, pm_ref):
    x = x_ref[0]
    logits = jax.lax.dot_general(
        w_ref[...], x, (((1,), (1,)), ((), ())),
        preferred_element_type=jnp.float32) + b_ref[...]
    m = jnp.max(logits, axis=0, keepdims=True)
    p = jnp.exp(logits - m)
    probs = p / jnp.sum(p, axis=0, keepdims=True)

    iota_e = jax.lax.broadcasted_iota(jnp.int32, probs.shape, 0)
    mask = jnp.zeros(probs.shape, jnp.bool_)
    wp = probs
    for _ in range(K):
        mx = jnp.max(wp, axis=0, keepdims=True)
        sel_idx = jnp.min(jnp.where(wp == mx, iota_e, E), axis=0,
                          keepdims=True)
        sel = iota_e == sel_idx
        mask = jnp.logical_or(mask, sel)
        wp = jnp.where(sel, -jnp.inf, wp)
    pm_ref[0] = jnp.where(mask, probs, 0.0)


def _sum_kernel(pm_ref, ex_ref, out_ref):
    pm = pm_ref[0]
    acc = pm[:, 0:1] * ex_ref[0, 0]
    for e in range(1, E):
        acc = acc + pm[:, e:e + 1] * ex_ref[0, e]
    out_ref[0] = acc


@jax.jit
def kernel(x, experts, gate_w, gate_b):
    N, C, H, W = x.shape
    S = H * W
    xs = jnp.transpose(x, (0, 2, 3, 1)).reshape(N, S, C)
    exs = experts.reshape(N, E, S, D)
    b2 = gate_b.reshape(E, 1)

    pm0 = pl.pallas_call(
        _gate_kernel,
        grid=(N,),
        in_specs=[
            pl.BlockSpec((1, S, C), lambda n: (n, 0, 0)),
            pl.BlockSpec((E, C), lambda n: (0, 0)),
            pl.BlockSpec((E, 1), lambda n: (0, 0)),
        ],
        out_specs=pl.BlockSpec((1, E, S), lambda n: (n, 0, 0)),
        out_shape=jax.ShapeDtypeStruct((N, E, S), jnp.float32),
    )(xs, gate_w, b2)

    pmt = jnp.transpose(pm0, (0, 2, 1))

    grid = (N, S // S_BLK)
    out = pl.pallas_call(
        _sum_kernel,
        grid=grid,
        in_specs=[
            pl.BlockSpec((1, S_BLK, E), lambda n, s: (n, s, 0)),
            pl.BlockSpec((1, E, S_BLK, D), lambda n, s: (n, 0, s, 0)),
        ],
        out_specs=pl.BlockSpec((1, S_BLK, D), lambda n, s: (n, s, 0)),
        out_shape=jax.ShapeDtypeStruct((N, S, D), jnp.float32),
    )(pmt, exs)
    return out.reshape(N, H, W, D)
